# Initial kernel scaffold; baseline (speedup 1.0000x reference)
import jax
import jax.numpy as jnp
from jax.experimental import pallas as pl


def kernel(atom_features, edge_index, edge_features, num_atoms, W_i, b_i, W_h, b_h, W_o, b_o):
    raise NotImplementedError("write your pallas kernel here")


# speedup vs baseline: 1.7635x; 1.7635x over previous
import functools

import jax
import jax.numpy as jnp
from jax import lax
from jax.experimental import pallas as pl
from jax.experimental.pallas import tpu as pltpu
from jax.experimental.pallas import tpu_sc as plsc

NC = 2
NS = 16
NW = NC * NS
HP = 384
CNT = 300


def _mesh():
    return plsc.VectorSubcoreMesh(core_axis_name="c", subcore_axis_name="s")


def _sc_gather_rows(table, idx, chunk=200):
    n, w = table.shape
    e = idx.shape[0]
    per_w = e // NW
    assert e % NW == 0 and per_w % chunk == 0 and chunk % 8 == 0
    assert w % 128 == 0
    nch = per_w // chunk

    @functools.partial(
        pl.kernel, mesh=_mesh(),
        out_type=jax.ShapeDtypeStruct((e, w), jnp.float32),
        scratch_types=[
            pltpu.VMEM((chunk,), jnp.int32),
            pltpu.VMEM((chunk, w), jnp.float32),
            pltpu.SemaphoreType.DMA,
        ],
    )
    def k(table_hbm, idx_hbm, out_hbm, idx_v, rows_v, sem):
        wid = lax.axis_index("s") * NC + lax.axis_index("c")
        base = wid * per_w

        def body(j, carry):
            off = pl.multiple_of(base + j * chunk, 8)
            pltpu.sync_copy(idx_hbm.at[pl.ds(off, chunk)], idx_v)
            pltpu.async_copy(table_hbm.at[idx_v], rows_v, sem).wait()
            pltpu.sync_copy(rows_v, out_hbm.at[pl.ds(off, chunk)])
            return carry

        lax.fori_loop(0, nch, body, 0)

    return k(table, idx)


def _sc_scatter_add_rows(values, idx, n_rows, chunk=80):
    e, w = values.shape
    per_s = e // NS
    ngrp = w // 128
    stride = 40 * NS
    acc_r = -(-n_rows // stride) * stride
    assert e % NS == 0 and per_s % chunk == 0 and chunk % 8 == 0
    assert w % 128 == 0 and n_rows % 40 == 0
    nch = per_s // chunk
    zeros = jnp.zeros((acc_r, 128), dtype=jnp.float32)

    @functools.partial(
        pl.kernel, mesh=_mesh(),
        out_type=jax.ShapeDtypeStruct((n_rows, w), jnp.float32),
        scratch_types=[
            pltpu.VMEM((chunk,), jnp.int32),
            pltpu.VMEM((chunk, 128), jnp.float32),
            pltpu.VMEM_SHARED((acc_r, 128), jnp.float32),
            pltpu.SemaphoreType.DMA,
        ],
    )
    def k(val_hbm, idx_hbm, zero_hbm, out_hbm, idx_v, rows_v, acc_sh, sem):
        c = lax.axis_index("c")
        s = lax.axis_index("s")
        for g in range(ngrp):

            @pl.when(g % NC == c)
            def _grp():
                for j in range(acc_r // stride):
                    off = pl.multiple_of(s * 40 + j * stride, 40)
                    pltpu.sync_copy(zero_hbm.at[pl.ds(off, 40)],
                                    acc_sh.at[pl.ds(off, 40)])
                plsc.subcore_barrier()

                def body(jj, carry):
                    off = pl.multiple_of(s * per_s + jj * chunk, 8)
                    pltpu.sync_copy(idx_hbm.at[pl.ds(off, chunk)], idx_v)
                    pltpu.sync_copy(
                        val_hbm.at[pl.ds(off, chunk),
                                   pl.ds(g * 128, 128)], rows_v)
                    pltpu.sync_copy(rows_v, acc_sh.at[idx_v], add=True)
                    return carry

                lax.fori_loop(0, nch, body, 0)
                plsc.subcore_barrier()
                for j in range(acc_r // stride):
                    off = pl.multiple_of(s * 40 + j * stride, 40)

                    @pl.when(off < n_rows)
                    def _cp():
                        pltpu.sync_copy(
                            acc_sh.at[pl.ds(off, 40)],
                            out_hbm.at[pl.ds(off, 40), pl.ds(g * 128, 128)])
                plsc.subcore_barrier()

    return k(values, idx, zeros)


def _tc_node_table(x, wt, cnt, blk=1000):
    n, f = x.shape

    def body(x_ref, w_ref, c_ref, o_ref):
        mm = jnp.dot(x_ref[...], w_ref[...],
                     preferred_element_type=jnp.float32)
        c1 = c_ref[...][:, :1]
        o_ref[...] = jnp.concatenate(
            [mm[:, :CNT], c1, mm[:, CNT + 1:]], axis=1)

    return pl.pallas_call(
        body,
        grid=(n // blk,),
        in_specs=[
            pl.BlockSpec((blk, f), lambda i: (i, 0)),
            pl.BlockSpec((f, HP), lambda i: (0, 0)),
            pl.BlockSpec((blk, 128), lambda i: (i, 0)),
        ],
        out_specs=pl.BlockSpec((blk, HP), lambda i: (i, 0)),
        out_shape=jax.ShapeDtypeStruct((n, HP), jnp.float32),
    )(x, wt, cnt)


def _tc_wi_combine(gaw, ef, wiet, b_i, b_h, sl, blk=640):
    e = gaw.shape[0]
    f = ef.shape[1]

    def body(g_ref, e_ref, w_ref, bi_ref, bh_ref, s_ref, wi_ref, m_ref):
        g = g_ref[...]
        wi = g + jnp.dot(e_ref[...], w_ref[...],
                         preferred_element_type=jnp.float32) + bi_ref[...]
        nb = (g[:, CNT:CNT + 1] - s_ref[...]) > 0.0
        wi_ref[...] = wi
        m_ref[...] = jnp.maximum(jnp.where(nb, wi + bh_ref[...], wi), 0.0)

    return pl.pallas_call(
        body,
        grid=(e // blk,),
        in_specs=[
            pl.BlockSpec((blk, HP), lambda i: (i, 0)),
            pl.BlockSpec((blk, f), lambda i: (i, 0)),
            pl.BlockSpec((f, HP), lambda i: (0, 0)),
            pl.BlockSpec((1, HP), lambda i: (0, 0)),
            pl.BlockSpec((1, HP), lambda i: (0, 0)),
            pl.BlockSpec((blk, 1), lambda i: (i, 0)),
        ],
        out_specs=[
            pl.BlockSpec((blk, HP), lambda i: (i, 0)),
            pl.BlockSpec((blk, HP), lambda i: (i, 0)),
        ],
        out_shape=[
            jax.ShapeDtypeStruct((e, HP), jnp.float32),
            jax.ShapeDtypeStruct((e, HP), jnp.float32),
        ],
    )(gaw, ef, wiet, b_i, b_h, sl)


def _tc_edge_step(wi, g, m, sl, wht, b_h, blk=640):
    e = wi.shape[0]

    def body(wi_ref, g_ref, m_ref, s_ref, w_ref, bh_ref, o_ref):
        wi = wi_ref[...]
        agg = g_ref[...] - s_ref[...] * m_ref[...]
        wh = jnp.dot(agg, w_ref[...], preferred_element_type=jnp.float32)
        with_h = wi + wh + bh_ref[...]
        nb = (wi[:, CNT:CNT + 1] - s_ref[...]) > 0.0
        o_ref[...] = jnp.maximum(jnp.where(nb, with_h, wi), 0.0)

    return pl.pallas_call(
        body,
        grid=(e // blk,),
        in_specs=[
            pl.BlockSpec((blk, HP), lambda i: (i, 0)),
            pl.BlockSpec((blk, HP), lambda i: (i, 0)),
            pl.BlockSpec((blk, HP), lambda i: (i, 0)),
            pl.BlockSpec((blk, 1), lambda i: (i, 0)),
            pl.BlockSpec((HP, HP), lambda i: (0, 0)),
            pl.BlockSpec((1, HP), lambda i: (0, 0)),
        ],
        out_specs=pl.BlockSpec((blk, HP), lambda i: (i, 0)),
        out_shape=jax.ShapeDtypeStruct((e, HP), jnp.float32),
    )(wi, g, m, sl, wht, b_h)


def _tc_readout(atom, in_agg, cnt, woat, womt, b_o, blk=1000):
    n, f = atom.shape
    h = woat.shape[1]
    nblk = n // blk

    def body(a_ref, g_ref, c_ref, wa_ref, wm_ref, bo_ref, o_ref):
        i = pl.program_id(0)
        reps = (jnp.dot(a_ref[...], wa_ref[...],
                        preferred_element_type=jnp.float32)
                + jnp.dot(g_ref[...], wm_ref[...],
                          preferred_element_type=jnp.float32)
                + bo_ref[...])
        reps = jnp.maximum(reps, 0.0)
        reps = jnp.where(c_ref[...][:, :1] > 0.0, reps, 0.0)
        part = jnp.broadcast_to(jnp.sum(reps, axis=0, keepdims=True), (8, h))

        @pl.when(i == 0)
        def _init():
            o_ref[...] = part

        @pl.when(i > 0)
        def _acc():
            o_ref[...] = o_ref[...] + part

        @pl.when(i == nblk - 1)
        def _fin():
            o_ref[...] = o_ref[...] * (1.0 / n)

    return pl.pallas_call(
        body,
        grid=(nblk,),
        in_specs=[
            pl.BlockSpec((blk, f), lambda i: (i, 0)),
            pl.BlockSpec((blk, HP), lambda i: (i, 0)),
            pl.BlockSpec((blk, 128), lambda i: (i, 0)),
            pl.BlockSpec((f, h), lambda i: (0, 0)),
            pl.BlockSpec((HP, h), lambda i: (0, 0)),
            pl.BlockSpec((1, h), lambda i: (0, 0)),
        ],
        out_specs=pl.BlockSpec((8, h), lambda i: (0, 0)),
        out_shape=jax.ShapeDtypeStruct((8, h), jnp.float32),
    )(atom, in_agg, cnt, woat, womt, b_o)


def _pad_cols(a, width):
    return jnp.pad(a, ((0, 0), (0, width - a.shape[1])))


def kernel(atom_features, edge_index, edge_features, num_atoms,
           W_i, b_i, W_h, b_h, W_o, b_o):
    n, f = atom_features.shape
    e = edge_index.shape[1]
    h = W_h.shape[0]
    depth = 3

    src = edge_index[0]
    tgt = edge_index[1]
    sl = (src == tgt).astype(jnp.float32)[:, None]
    ones128 = jnp.ones((e, 128), dtype=jnp.float32)

    wi_nt = _pad_cols(W_i[:, :f].T, HP)
    wi_et = _pad_cols(W_i[:, f:].T, HP)
    wht = _pad_cols(jnp.pad(W_h.T, ((0, HP - h), (0, 0))), HP)
    woat = W_o[:, :f].T
    womt = jnp.pad(W_o[:, f:].T, ((0, HP - h), (0, 0)))
    b_i2 = _pad_cols(b_i[None, :], HP)
    b_h2 = _pad_cols(b_h[None, :], HP)
    b_o2 = b_o[None, :]

    cnt_src = _sc_scatter_add_rows(ones128, src, n)
    cnt_tgt = _sc_scatter_add_rows(ones128, tgt, n)

    aw = _tc_node_table(atom_features, wi_nt, cnt_src)
    gaw = _sc_gather_rows(aw, tgt)

    wi_out, messages = _tc_wi_combine(gaw, edge_features, wi_et,
                                      b_i2, b_h2, sl)

    for _ in range(depth - 1):
        seg = _sc_scatter_add_rows(messages, src, n)
        g = _sc_gather_rows(seg, tgt)
        messages = _tc_edge_step(wi_out, g, messages, sl, wht, b_h2)

    in_agg = _sc_scatter_add_rows(messages, tgt, n)
    out = _tc_readout(atom_features, in_agg, cnt_tgt, woat, womt, b_o2)
    return out[0] + 0.0 * (jnp.asarray(num_atoms, dtype=jnp.float32) - n)


# speedup vs baseline: 2.1548x; 1.2219x over previous
import functools

import jax
import jax.numpy as jnp
from jax import lax
from jax.experimental import pallas as pl
from jax.experimental.pallas import tpu as pltpu
from jax.experimental.pallas import tpu_sc as plsc

NC = 2
NS = 16
NW = NC * NS
HP = 384
CNT = 300


def _mesh():
    return plsc.VectorSubcoreMesh(core_axis_name="c", subcore_axis_name="s")


def _sc_gather_rows(table, idx, chunk=200):
    n, w = table.shape
    e = idx.shape[0]
    per_w = e // NW
    assert e % NW == 0 and per_w % chunk == 0 and chunk % 8 == 0
    assert w % 128 == 0
    nch = per_w // chunk

    @functools.partial(
        pl.kernel, mesh=_mesh(),
        out_type=jax.ShapeDtypeStruct((e, w), jnp.float32),
        scratch_types=[
            pltpu.VMEM((chunk,), jnp.int32),
            pltpu.VMEM((chunk, w), jnp.float32),
            pltpu.SemaphoreType.DMA,
        ],
    )
    def k(table_hbm, idx_hbm, out_hbm, idx_v, rows_v, sem):
        wid = lax.axis_index("s") * NC + lax.axis_index("c")
        base = wid * per_w

        def body(j, carry):
            off = pl.multiple_of(base + j * chunk, 8)
            pltpu.sync_copy(idx_hbm.at[pl.ds(off, chunk)], idx_v)
            pltpu.async_copy(table_hbm.at[idx_v], rows_v, sem).wait()
            pltpu.sync_copy(rows_v, out_hbm.at[pl.ds(off, chunk)])
            return carry

        lax.fori_loop(0, nch, body, 0)

    return k(table, idx)


def _sc_scatter_add_rows(values, idx, n_rows, chunk=200, const_val=False):
    e = idx.shape[0]
    w = 128 if const_val else values.shape[1]
    per_s = e // NS
    ngrp = w // 128
    stride = 40 * NS
    acc_r = -(-n_rows // stride) * stride
    assert e % NS == 0 and per_s % chunk == 0 and chunk % 8 == 0
    assert w % 128 == 0 and n_rows % 40 == 0
    nch = per_s // chunk
    zeros = jnp.zeros((acc_r, 128), dtype=jnp.float32)

    @functools.partial(
        pl.kernel, mesh=_mesh(),
        out_type=jax.ShapeDtypeStruct((n_rows, w), jnp.float32),
        scratch_types=[
            pltpu.VMEM((chunk,), jnp.int32),
            pltpu.VMEM((chunk, 128), jnp.float32),
            pltpu.VMEM_SHARED((acc_r, 128), jnp.float32),
            pltpu.SemaphoreType.DMA,
        ],
    )
    def k(val_hbm, idx_hbm, zero_hbm, out_hbm, idx_v, rows_v, acc_sh, sem):
        c = lax.axis_index("c")
        s = lax.axis_index("s")
        for g in range(ngrp):

            @pl.when(g % NC == c)
            def _grp():
                if const_val:
                    pltpu.sync_copy(val_hbm, rows_v)
                for j in range(acc_r // stride):
                    off = pl.multiple_of(s * 40 + j * stride, 40)
                    pltpu.sync_copy(zero_hbm.at[pl.ds(off, 40)],
                                    acc_sh.at[pl.ds(off, 40)])
                plsc.subcore_barrier()

                def body(jj, carry):
                    off = pl.multiple_of(s * per_s + jj * chunk, 8)
                    pltpu.sync_copy(idx_hbm.at[pl.ds(off, chunk)], idx_v)
                    if not const_val:
                        pltpu.sync_copy(
                            val_hbm.at[pl.ds(off, chunk),
                                       pl.ds(g * 128, 128)], rows_v)
                    pltpu.sync_copy(rows_v, acc_sh.at[idx_v], add=True)
                    return carry

                lax.fori_loop(0, nch, body, 0)
                plsc.subcore_barrier()
                for j in range(acc_r // stride):
                    off = pl.multiple_of(s * 40 + j * stride, 40)

                    @pl.when(off < n_rows)
                    def _cp():
                        pltpu.sync_copy(
                            acc_sh.at[pl.ds(off, 40)],
                            out_hbm.at[pl.ds(off, 40), pl.ds(g * 128, 128)])
                plsc.subcore_barrier()

    return k(values, idx, zeros)


def _tc_node_table(x, wt, cnt, blk=1000):
    n, f = x.shape

    def body(x_ref, w_ref, c_ref, o_ref):
        mm = jnp.dot(x_ref[...], w_ref[...],
                     preferred_element_type=jnp.float32)
        c1 = c_ref[...][:, :1]
        o_ref[...] = jnp.concatenate(
            [mm[:, :CNT], c1, mm[:, CNT + 1:]], axis=1)

    return pl.pallas_call(
        body,
        grid=(n // blk,),
        in_specs=[
            pl.BlockSpec((blk, f), lambda i: (i, 0)),
            pl.BlockSpec((f, HP), lambda i: (0, 0)),
            pl.BlockSpec((blk, 128), lambda i: (i, 0)),
        ],
        out_specs=pl.BlockSpec((blk, HP), lambda i: (i, 0)),
        out_shape=jax.ShapeDtypeStruct((n, HP), jnp.float32),
    )(x, wt, cnt)


def _tc_wi_combine(gaw, ef, wiet, b_i, b_h, sl, blk=640):
    e = gaw.shape[0]
    f = ef.shape[1]

    def body(g_ref, e_ref, w_ref, bi_ref, bh_ref, s_ref, wi_ref, m_ref):
        g = g_ref[...]
        wi = g + jnp.dot(e_ref[...], w_ref[...],
                         preferred_element_type=jnp.float32) + bi_ref[...]
        nb = (g[:, CNT:CNT + 1] - s_ref[...]) > 0.0
        wi_ref[...] = wi
        m_ref[...] = jnp.maximum(jnp.where(nb, wi + bh_ref[...], wi), 0.0)

    return pl.pallas_call(
        body,
        grid=(e // blk,),
        in_specs=[
            pl.BlockSpec((blk, HP), lambda i: (i, 0)),
            pl.BlockSpec((blk, f), lambda i: (i, 0)),
            pl.BlockSpec((f, HP), lambda i: (0, 0)),
            pl.BlockSpec((1, HP), lambda i: (0, 0)),
            pl.BlockSpec((1, HP), lambda i: (0, 0)),
            pl.BlockSpec((blk, 1), lambda i: (i, 0)),
        ],
        out_specs=[
            pl.BlockSpec((blk, HP), lambda i: (i, 0)),
            pl.BlockSpec((blk, HP), lambda i: (i, 0)),
        ],
        out_shape=[
            jax.ShapeDtypeStruct((e, HP), jnp.float32),
            jax.ShapeDtypeStruct((e, HP), jnp.float32),
        ],
    )(gaw, ef, wiet, b_i, b_h, sl)


def _tc_edge_step(wi, g, m, sl, wht, b_h, blk=640):
    e = wi.shape[0]

    def body(wi_ref, g_ref, m_ref, s_ref, w_ref, bh_ref, o_ref):
        wi = wi_ref[...]
        agg = g_ref[...] - s_ref[...] * m_ref[...]
        wh = jnp.dot(agg, w_ref[...], preferred_element_type=jnp.float32)
        with_h = wi + wh + bh_ref[...]
        nb = (wi[:, CNT:CNT + 1] - s_ref[...]) > 0.0
        o_ref[...] = jnp.maximum(jnp.where(nb, with_h, wi), 0.0)

    return pl.pallas_call(
        body,
        grid=(e // blk,),
        in_specs=[
            pl.BlockSpec((blk, HP), lambda i: (i, 0)),
            pl.BlockSpec((blk, HP), lambda i: (i, 0)),
            pl.BlockSpec((blk, HP), lambda i: (i, 0)),
            pl.BlockSpec((blk, 1), lambda i: (i, 0)),
            pl.BlockSpec((HP, HP), lambda i: (0, 0)),
            pl.BlockSpec((1, HP), lambda i: (0, 0)),
        ],
        out_specs=pl.BlockSpec((blk, HP), lambda i: (i, 0)),
        out_shape=jax.ShapeDtypeStruct((e, HP), jnp.float32),
    )(wi, g, m, sl, wht, b_h)


def _tc_readout(atom, in_agg, cnt, woat, womt, b_o, blk=1000):
    n, f = atom.shape
    h = woat.shape[1]
    nblk = n // blk

    def body(a_ref, g_ref, c_ref, wa_ref, wm_ref, bo_ref, o_ref):
        i = pl.program_id(0)
        reps = (jnp.dot(a_ref[...], wa_ref[...],
                        preferred_element_type=jnp.float32)
                + jnp.dot(g_ref[...], wm_ref[...],
                          preferred_element_type=jnp.float32)
                + bo_ref[...])
        reps = jnp.maximum(reps, 0.0)
        reps = jnp.where(c_ref[...][:, :1] > 0.0, reps, 0.0)
        part = jnp.broadcast_to(jnp.sum(reps, axis=0, keepdims=True), (8, h))

        @pl.when(i == 0)
        def _init():
            o_ref[...] = part

        @pl.when(i > 0)
        def _acc():
            o_ref[...] = o_ref[...] + part

        @pl.when(i == nblk - 1)
        def _fin():
            o_ref[...] = o_ref[...] * (1.0 / n)

    return pl.pallas_call(
        body,
        grid=(nblk,),
        in_specs=[
            pl.BlockSpec((blk, f), lambda i: (i, 0)),
            pl.BlockSpec((blk, HP), lambda i: (i, 0)),
            pl.BlockSpec((blk, 128), lambda i: (i, 0)),
            pl.BlockSpec((f, h), lambda i: (0, 0)),
            pl.BlockSpec((HP, h), lambda i: (0, 0)),
            pl.BlockSpec((1, h), lambda i: (0, 0)),
        ],
        out_specs=pl.BlockSpec((8, h), lambda i: (0, 0)),
        out_shape=jax.ShapeDtypeStruct((8, h), jnp.float32),
    )(atom, in_agg, cnt, woat, womt, b_o)


def _pad_cols(a, width):
    return jnp.pad(a, ((0, 0), (0, width - a.shape[1])))


def kernel(atom_features, edge_index, edge_features, num_atoms,
           W_i, b_i, W_h, b_h, W_o, b_o):
    n, f = atom_features.shape
    e = edge_index.shape[1]
    h = W_h.shape[0]
    depth = 3

    src = edge_index[0]
    tgt = edge_index[1]
    sl = (src == tgt).astype(jnp.float32)[:, None]
    ones128 = jnp.ones((200, 128), dtype=jnp.float32)

    wi_nt = _pad_cols(W_i[:, :f].T, HP)
    wi_et = _pad_cols(W_i[:, f:].T, HP)
    wht = _pad_cols(jnp.pad(W_h.T, ((0, HP - h), (0, 0))), HP)
    woat = W_o[:, :f].T
    womt = jnp.pad(W_o[:, f:].T, ((0, HP - h), (0, 0)))
    b_i2 = _pad_cols(b_i[None, :], HP)
    b_h2 = _pad_cols(b_h[None, :], HP)
    b_o2 = b_o[None, :]

    cnt_src = _sc_scatter_add_rows(ones128, src, n, const_val=True)
    cnt_tgt = _sc_scatter_add_rows(ones128, tgt, n, const_val=True)

    aw = _tc_node_table(atom_features, wi_nt, cnt_src)
    gaw = _sc_gather_rows(aw, tgt)

    wi_out, messages = _tc_wi_combine(gaw, edge_features, wi_et,
                                      b_i2, b_h2, sl)

    for _ in range(depth - 1):
        seg = _sc_scatter_add_rows(messages, src, n)
        g = _sc_gather_rows(seg, tgt)
        messages = _tc_edge_step(wi_out, g, messages, sl, wht, b_h2)

    in_agg = _sc_scatter_add_rows(messages, tgt, n)
    out = _tc_readout(atom_features, in_agg, cnt_tgt, woat, womt, b_o2)
    return out[0] + 0.0 * (jnp.asarray(num_atoms, dtype=jnp.float32) - n)
